# Initial kernel scaffold; baseline (speedup 1.0000x reference)
import jax
import jax.numpy as jnp
from jax.experimental import pallas as pl


def kernel(x, table):
    raise NotImplementedError("write your pallas kernel here")


# speedup vs baseline: 3.9315x; 3.9315x over previous
import functools

import jax
import jax.numpy as jnp
from jax import lax
from jax.experimental import pallas as pl
from jax.experimental.pallas import tpu as pltpu
from jax.experimental.pallas import tpu_sc as plsc

ROWS = 1000000
DIM = 32
N_HASH = 4
HASH_A = (2654435761, 2246822519, 3266489917, 668265263)
HASH_B = (374761393, 3550635116, 4251993797, 1640531527)

NC, NS, L = 2, 16, 16
NW = NC * NS
BATCH = 425984
BPW = BATCH // NW
C = 256
NCHUNK = BPW // C
IW = 128
NIDX = (N_HASH * C) // IW


def kernel(x, table):
    mesh = plsc.VectorSubcoreMesh(
        core_axis_name="c", subcore_axis_name="s",
        num_cores=NC, num_subcores=NS)

    @functools.partial(
        pl.kernel,
        out_type=jax.ShapeDtypeStruct((BATCH, DIM), jnp.float32),
        mesh=mesh,
        scratch_types=[
            pltpu.VMEM((C,), jnp.int32),
            pltpu.VMEM((NIDX, IW), jnp.int32),
            pltpu.VMEM((N_HASH * C, DIM), jnp.float32),
            pltpu.VMEM((C, DIM), jnp.float32),
            pltpu.SemaphoreType.DMA,
        ],
        compiler_params=pltpu.CompilerParams(use_tc_tiling_on_sc=False),
    )
    def run(x_hbm, table_hbm, out_hbm, xv, idxv, rows, outv, sem):
        wid = lax.axis_index("s") * NC + lax.axis_index("c")
        base = wid * BPW

        @pl.loop(0, NCHUNK)
        def chunk(g):
            off = base + g * C
            pltpu.sync_copy(x_hbm.at[pl.ds(off, C)], xv)
            for i in range(C // L):
                xu = plsc.bitcast(xv[pl.ds(i * L, L)], jnp.uint32)
                for h in range(N_HASH):
                    hv = (xu * jnp.uint32(HASH_A[h]) + jnp.uint32(HASH_B[h])
                          ) % jnp.uint32(ROWS)
                    p = h * C + i * L
                    idxv[p // IW, pl.ds(p % IW, L)] = plsc.bitcast(
                        hv, jnp.int32)
            cps = [
                pltpu.async_copy(
                    table_hbm.at[idxv.at[j]],
                    rows.at[pl.ds(j * IW, IW)], sem)
                for j in range(NIDX)
            ]
            for cp in cps:
                cp.wait()

            @plsc.parallel_loop(0, C, unroll=4)
            def acc(c):
                for d in range(0, DIM, L):
                    s = (rows[c, pl.ds(d, L)]
                         + rows[C + c, pl.ds(d, L)]
                         + rows[2 * C + c, pl.ds(d, L)]
                         + rows[3 * C + c, pl.ds(d, L)])
                    outv[c, pl.ds(d, L)] = s * jnp.float32(0.25)

            pltpu.sync_copy(outv, out_hbm.at[pl.ds(off, C), :])

    return run(x, table)


# speedup vs baseline: 4.4365x; 1.1285x over previous
import functools

import jax
import jax.numpy as jnp
from jax import lax
from jax.experimental import pallas as pl
from jax.experimental.pallas import tpu as pltpu
from jax.experimental.pallas import tpu_sc as plsc

ROWS = 1000000
DIM = 32
N_HASH = 4
HASH_A = (2654435761, 2246822519, 3266489917, 668265263)
HASH_B = (374761393, 3550635116, 4251993797, 1640531527)

NC, NS, L = 2, 16, 16
NW = NC * NS
BATCH = 425984
BPW = BATCH // NW
C = 256
NCHUNK = BPW // C
IW = 128
NIDX = (N_HASH * C) // IW


def kernel(x, table):
    mesh = plsc.VectorSubcoreMesh(
        core_axis_name="c", subcore_axis_name="s",
        num_cores=NC, num_subcores=NS)

    @functools.partial(
        pl.kernel,
        out_type=jax.ShapeDtypeStruct((BATCH, DIM), jnp.float32),
        mesh=mesh,
        scratch_types=[
            pltpu.VMEM((2, C), jnp.int32),
            pltpu.VMEM((2, NIDX, IW), jnp.int32),
            pltpu.VMEM((2, N_HASH * C, DIM), jnp.float32),
            pltpu.VMEM((2, C, DIM), jnp.float32),
            pltpu.SemaphoreType.DMA((2,)),
            pltpu.SemaphoreType.DMA((2,)),
        ],
        compiler_params=pltpu.CompilerParams(use_tc_tiling_on_sc=False),
    )
    def run(x_hbm, table_hbm, out_hbm, xv, idxv, rows, outv, gsem, osem):
        wid = lax.axis_index("s") * NC + lax.axis_index("c")
        base = wid * BPW

        def fire(g, b):
            off = base + g * C
            pltpu.sync_copy(x_hbm.at[pl.ds(off, C)], xv.at[b])
            for i in range(C // L):
                xu = plsc.bitcast(xv[b, pl.ds(i * L, L)], jnp.uint32)
                for h in range(N_HASH):
                    hv = (xu * jnp.uint32(HASH_A[h]) + jnp.uint32(HASH_B[h])
                          ) % jnp.uint32(ROWS)
                    p = h * C + i * L
                    idxv[b, p // IW, pl.ds(p % IW, L)] = plsc.bitcast(
                        hv, jnp.int32)
            for j in range(NIDX):
                pltpu.async_copy(
                    table_hbm.at[idxv.at[b, j]],
                    rows.at[b, pl.ds(j * IW, IW)],
                    gsem.at[b])

        def drain_and_combine(g, b):
            for j in range(NIDX):
                pltpu.make_async_copy(
                    table_hbm.at[idxv.at[b, j]],
                    rows.at[b, pl.ds(j * IW, IW)],
                    gsem.at[b]).wait()

            @pl.when(g >= 2)
            def _():
                pltpu.make_async_copy(
                    outv.at[b], out_hbm.at[pl.ds(0, C), :],
                    osem.at[b]).wait()

            @plsc.parallel_loop(0, C, unroll=4)
            def acc(c):
                for d in range(0, DIM, L):
                    s = (rows[b, c, pl.ds(d, L)]
                         + rows[b, C + c, pl.ds(d, L)]
                         + rows[b, 2 * C + c, pl.ds(d, L)]
                         + rows[b, 3 * C + c, pl.ds(d, L)])
                    outv[b, c, pl.ds(d, L)] = s * jnp.float32(0.25)

            off = base + g * C
            pltpu.async_copy(
                outv.at[b], out_hbm.at[pl.ds(off, C), :], osem.at[b])

        fire(0, 0)

        @pl.loop(0, NCHUNK, step=2)
        def chunk(g):
            for b in range(2):
                cur = g + b
                nxt = cur + 1

                @pl.when(nxt < NCHUNK)
                def _():
                    fire(nxt, 1 - b)

                drain_and_combine(cur, b)

        for b in range(2):
            pltpu.make_async_copy(
                outv.at[b], out_hbm.at[pl.ds(0, C), :], osem.at[b]).wait()

    return run(x, table)


# speedup vs baseline: 4.4391x; 1.0006x over previous
import functools

import jax
import jax.numpy as jnp
from jax import lax
from jax.experimental import pallas as pl
from jax.experimental.pallas import tpu as pltpu
from jax.experimental.pallas import tpu_sc as plsc

ROWS = 1000000
DIM = 32
N_HASH = 4
HASH_A = (2654435761, 2246822519, 3266489917, 668265263)
HASH_B = (374761393, 3550635116, 4251993797, 1640531527)

NC, NS, L = 2, 16, 16
NW = NC * NS
BATCH = 425984
BPW = BATCH // NW
C = 256
NCHUNK = BPW // C
IW = 128
NIDX = (N_HASH * C) // IW


def kernel(x, table):
    mesh = plsc.VectorSubcoreMesh(
        core_axis_name="c", subcore_axis_name="s",
        num_cores=NC, num_subcores=NS)

    @functools.partial(
        pl.kernel,
        out_type=jax.ShapeDtypeStruct((BATCH * DIM,), jnp.float32),
        mesh=mesh,
        scratch_types=[
            pltpu.VMEM((2, C), jnp.int32),
            pltpu.VMEM((2, NIDX, IW), jnp.int32),
            pltpu.VMEM((2, N_HASH * C, DIM), jnp.float32),
            pltpu.VMEM((2, C * DIM), jnp.float32),
            pltpu.SemaphoreType.DMA((2,)),
            pltpu.SemaphoreType.DMA((2,)),
        ],
        compiler_params=pltpu.CompilerParams(use_tc_tiling_on_sc=False),
    )
    def run(x_hbm, table_hbm, out_hbm, xv, idxv, rows, outv, gsem, osem):
        wid = lax.axis_index("s") * NC + lax.axis_index("c")
        base = wid * BPW

        def fire(g, b):
            off = base + g * C
            pltpu.sync_copy(x_hbm.at[pl.ds(off, C)], xv.at[b])
            for i in range(C // L):
                xu = plsc.bitcast(xv[b, pl.ds(i * L, L)], jnp.uint32)
                for h in range(N_HASH):
                    hv = (xu * jnp.uint32(HASH_A[h]) + jnp.uint32(HASH_B[h])
                          ) % jnp.uint32(ROWS)
                    p = h * C + i * L
                    idxv[b, p // IW, pl.ds(p % IW, L)] = plsc.bitcast(
                        hv, jnp.int32)
            for j in range(NIDX):
                pltpu.async_copy(
                    table_hbm.at[idxv.at[b, j]],
                    rows.at[b, pl.ds(j * IW, IW)],
                    gsem.at[b])

        def drain_and_combine(g, b):
            for j in range(NIDX):
                pltpu.make_async_copy(
                    table_hbm.at[idxv.at[b, j]],
                    rows.at[b, pl.ds(j * IW, IW)],
                    gsem.at[b]).wait()

            @pl.when(g >= 2)
            def _():
                pltpu.make_async_copy(
                    outv.at[b], out_hbm.at[pl.ds(0, C * DIM)],
                    osem.at[b]).wait()

            @plsc.parallel_loop(0, C, unroll=4)
            def acc(c):
                for d in range(0, DIM, L):
                    s = (rows[b, c, pl.ds(d, L)]
                         + rows[b, C + c, pl.ds(d, L)]
                         + rows[b, 2 * C + c, pl.ds(d, L)]
                         + rows[b, 3 * C + c, pl.ds(d, L)])
                    outv[b, pl.ds(c * DIM + d, L)] = s * jnp.float32(0.25)

            off = base + g * C
            pltpu.async_copy(
                outv.at[b], out_hbm.at[pl.ds(off * DIM, C * DIM)], osem.at[b])

        fire(0, 0)

        @pl.loop(0, NCHUNK, step=2)
        def chunk(g):
            for b in range(2):
                cur = g + b
                nxt = cur + 1

                @pl.when(nxt < NCHUNK)
                def _():
                    fire(nxt, 1 - b)

                drain_and_combine(cur, b)

        for b in range(2):
            pltpu.make_async_copy(
                outv.at[b], out_hbm.at[pl.ds(0, C * DIM)], osem.at[b]).wait()

    return run(x, table).reshape(BATCH, DIM)


# speedup vs baseline: 4.4478x; 1.0020x over previous
import functools

import jax
import jax.numpy as jnp
from jax import lax
from jax.experimental import pallas as pl
from jax.experimental.pallas import tpu as pltpu
from jax.experimental.pallas import tpu_sc as plsc

ROWS = 1000000
DIM = 32
N_HASH = 4
HASH_A = (2654435761, 2246822519, 3266489917, 668265263)
HASH_B = (374761393, 3550635116, 4251993797, 1640531527)

NC, NS, L = 2, 16, 16
NW = NC * NS
BATCH = 425984
BPW = BATCH // NW
C = 256
NCHUNK = BPW // C
IW = 128
NIDX = (N_HASH * C) // IW


def kernel(x, table):
    mesh = plsc.VectorSubcoreMesh(
        core_axis_name="c", subcore_axis_name="s",
        num_cores=NC, num_subcores=NS)

    @functools.partial(
        pl.kernel,
        out_type=jax.ShapeDtypeStruct((DIM, BATCH), jnp.float32),
        mesh=mesh,
        scratch_types=[
            pltpu.VMEM((2, C), jnp.int32),
            pltpu.VMEM((2, NIDX, IW), jnp.int32),
            pltpu.VMEM((2, N_HASH * C, DIM), jnp.float32),
            pltpu.VMEM((2, DIM, C), jnp.float32),
            pltpu.SemaphoreType.DMA((2,)),
            pltpu.SemaphoreType.DMA((2,)),
        ],
        compiler_params=pltpu.CompilerParams(
            use_tc_tiling_on_sc=False, needs_layout_passes=False),
    )
    def run(x_hbm, table_hbm, out_hbm, xv, idxv, rows, outv, gsem, osem):
        wid = lax.axis_index("s") * NC + lax.axis_index("c")
        base = wid * BPW

        def fire(g, b):
            off = base + g * C
            pltpu.sync_copy(x_hbm.at[pl.ds(off, C)], xv.at[b])
            for i in range(C // L):
                xu = plsc.bitcast(xv[b, pl.ds(i * L, L)], jnp.uint32)
                for h in range(N_HASH):
                    hv = (xu * jnp.uint32(HASH_A[h]) + jnp.uint32(HASH_B[h])
                          ) % jnp.uint32(ROWS)
                    p = h * C + i * L
                    idxv[b, p // IW, pl.ds(p % IW, L)] = plsc.bitcast(
                        hv, jnp.int32)
            for j in range(NIDX):
                pltpu.async_copy(
                    table_hbm.at[idxv.at[b, j]],
                    rows.at[b, pl.ds(j * IW, IW)],
                    gsem.at[b])

        def drain_and_combine(g, b):
            for j in range(NIDX):
                pltpu.make_async_copy(
                    table_hbm.at[idxv.at[b, j]],
                    rows.at[b, pl.ds(j * IW, IW)],
                    gsem.at[b]).wait()

            @pl.when(g >= 2)
            def _():
                pltpu.make_async_copy(
                    outv.at[b], out_hbm.at[:, pl.ds(0, C)],
                    osem.at[b]).wait()

            iota = lax.iota(jnp.int32, L)

            @plsc.parallel_loop(0, C, unroll=4)
            def acc(c):
                colv = jnp.zeros((L,), jnp.int32) + c
                for d in range(0, DIM, L):
                    s = (rows[b, c, pl.ds(d, L)]
                         + rows[b, C + c, pl.ds(d, L)]
                         + rows[b, 2 * C + c, pl.ds(d, L)]
                         + rows[b, 3 * C + c, pl.ds(d, L)])
                    plsc.store_scatter(outv.at[b], [iota + d, colv],
                                       s * jnp.float32(0.25))

            off = base + g * C
            pltpu.async_copy(
                outv.at[b], out_hbm.at[:, pl.ds(off, C)], osem.at[b])

        fire(0, 0)

        @pl.loop(0, NCHUNK, step=2)
        def chunk(g):
            for b in range(2):
                cur = g + b
                nxt = cur + 1

                @pl.when(nxt < NCHUNK)
                def _():
                    fire(nxt, 1 - b)

                drain_and_combine(cur, b)

        for b in range(2):
            pltpu.make_async_copy(
                outv.at[b], out_hbm.at[:, pl.ds(0, C)], osem.at[b]).wait()

    return run(x, table).T


# speedup vs baseline: 5.4687x; 1.2295x over previous
import functools

import jax
import jax.numpy as jnp
from jax import lax
from jax.experimental import pallas as pl
from jax.experimental.pallas import tpu as pltpu
from jax.experimental.pallas import tpu_sc as plsc

ROWS = 1000000
DIM = 32
N_HASH = 4
HASH_A = (2654435761, 2246822519, 3266489917, 668265263)
HASH_B = (374761393, 3550635116, 4251993797, 1640531527)

NC, NS, L = 2, 16, 16
NW = NC * NS
BATCH = 425984
BPW = BATCH // NW
C = 256
NCHUNK = BPW // C
IW = 128
NIDX = (N_HASH * C) // IW


def kernel(x, table):
    mesh = plsc.VectorSubcoreMesh(
        core_axis_name="c", subcore_axis_name="s",
        num_cores=NC, num_subcores=NS)

    @functools.partial(
        pl.kernel,
        out_type=jax.ShapeDtypeStruct((DIM, BATCH), jnp.float32),
        mesh=mesh,
        scratch_types=[
            pltpu.VMEM((2, C), jnp.int32),
            pltpu.VMEM((2, NIDX, IW), jnp.int32),
            pltpu.VMEM((2, N_HASH * C, DIM), jnp.float32),
            pltpu.VMEM((2, DIM, C + 1), jnp.float32),
            pltpu.SemaphoreType.DMA((2,)),
            pltpu.SemaphoreType.DMA((2,)),
        ],
        compiler_params=pltpu.CompilerParams(
            use_tc_tiling_on_sc=False, needs_layout_passes=False),
    )
    def run(x_hbm, table_hbm, out_hbm, xv, idxv, rows, outv, gsem, osem):
        wid = lax.axis_index("s") * NC + lax.axis_index("c")
        base = wid * BPW

        def fire(g, b):
            off = base + g * C
            pltpu.sync_copy(x_hbm.at[pl.ds(off, C)], xv.at[b])
            for i in range(C // L):
                xu = plsc.bitcast(xv[b, pl.ds(i * L, L)], jnp.uint32)
                for h in range(N_HASH):
                    hv = (xu * jnp.uint32(HASH_A[h]) + jnp.uint32(HASH_B[h])
                          ) % jnp.uint32(ROWS)
                    p = h * C + i * L
                    idxv[b, p // IW, pl.ds(p % IW, L)] = plsc.bitcast(
                        hv, jnp.int32)
            for j in range(NIDX):
                pltpu.async_copy(
                    table_hbm.at[idxv.at[b, j]],
                    rows.at[b, pl.ds(j * IW, IW)],
                    gsem.at[b])

        def drain_and_combine(g, b):
            for j in range(NIDX):
                pltpu.make_async_copy(
                    table_hbm.at[idxv.at[b, j]],
                    rows.at[b, pl.ds(j * IW, IW)],
                    gsem.at[b]).wait()

            @pl.when(g >= 2)
            def _():
                pltpu.make_async_copy(
                    outv.at[b, :, pl.ds(0, C)], out_hbm.at[:, pl.ds(0, C)],
                    osem.at[b]).wait()

            iota = lax.iota(jnp.int32, L)

            @plsc.parallel_loop(0, C, unroll=4)
            def acc(c):
                colv = jnp.zeros((L,), jnp.int32) + c
                for d in range(0, DIM, L):
                    s = (rows[b, c, pl.ds(d, L)]
                         + rows[b, C + c, pl.ds(d, L)]
                         + rows[b, 2 * C + c, pl.ds(d, L)]
                         + rows[b, 3 * C + c, pl.ds(d, L)])
                    plsc.store_scatter(outv.at[b], [iota + d, colv],
                                       s * jnp.float32(0.25))

            off = base + g * C
            pltpu.async_copy(
                outv.at[b, :, pl.ds(0, C)],
                out_hbm.at[:, pl.ds(off, C)], osem.at[b])

        fire(0, 0)

        @pl.loop(0, NCHUNK, step=2)
        def chunk(g):
            for b in range(2):
                cur = g + b
                nxt = cur + 1

                @pl.when(nxt < NCHUNK)
                def _():
                    fire(nxt, 1 - b)

                drain_and_combine(cur, b)

        for b in range(2):
            pltpu.make_async_copy(
                outv.at[b, :, pl.ds(0, C)], out_hbm.at[:, pl.ds(0, C)],
                osem.at[b]).wait()

    return run(x, table).T
